# baseline (speedup 1.0000x reference)
import functools

import jax
import jax.numpy as jnp
from jax import lax
from jax.experimental import pallas as pl
from jax.experimental.pallas import tpu as pltpu
from jax.experimental.pallas import tpu_sc as plsc

B = 16384
EMB = 50
BERT_DIM = 768
NCAT = 1000
NW = 32
BPW = B // NW
BC = 64
NBC = BPW // BC


def _sc_gather(users, items, ent0, user_table, news_table, entity_table,
               bert_table):
    mesh = plsc.VectorSubcoreMesh(core_axis_name="c", subcore_axis_name="s")
    out_type = (
        jax.ShapeDtypeStruct((B, EMB), jnp.float32),
        jax.ShapeDtypeStruct((B, EMB), jnp.float32),
        jax.ShapeDtypeStruct((B, EMB), jnp.float32),
        jax.ShapeDtypeStruct((B, BERT_DIM), jnp.float32),
    )

    @functools.partial(
        pl.kernel, mesh=mesh, out_type=out_type,
        scratch_types=[
            pltpu.VMEM((BPW, EMB), jnp.float32),
            pltpu.VMEM((BPW,), jnp.int32),
            pltpu.VMEM((BC, BERT_DIM), jnp.float32),
            pltpu.SemaphoreType.DMA,
            pltpu.SemaphoreType.DMA,
        ],
        compiler_params=pltpu.CompilerParams(needs_layout_passes=False),
    )
    def k(users_h, items_h, ent_h, user_t, news_t, ent_t, bert_t,
          out_user, out_news, out_ent, out_bert,
          rows_v, idx_v, row768_v, sem, bsem):
        wid = lax.axis_index("s") * 2 + lax.axis_index("c")
        base = wid * BPW
        for idx_h, tab, out in ((users_h, user_t, out_user),
                                (items_h, news_t, out_news),
                                (ent_h, ent_t, out_ent)):
            pltpu.sync_copy(idx_h.at[pl.ds(base, BPW)], idx_v)

            def fire(r, _):
                rr = jnp.full((16,), r, jnp.int32)
                iv = plsc.load_gather(idx_v, [rr])
                pltpu.async_copy(tab.at[pl.ds(jnp.max(iv), 1)],
                                 rows_v.at[pl.ds(r, 1)], sem)
                return 0

            lax.fori_loop(0, BPW, fire, 0)

            def drain(r, _):
                pltpu.make_async_copy(tab.at[pl.ds(0, 1)],
                                      rows_v.at[pl.ds(0, 1)], sem).wait()
                return 0

            lax.fori_loop(0, BPW, drain, 0)
            pltpu.sync_copy(rows_v, out.at[pl.ds(base, BPW)])
        pltpu.sync_copy(items_h.at[pl.ds(base, BPW)], idx_v)
        for c in range(NBC):
            pltpu.async_copy(
                bert_t.at[idx_v.at[pl.ds(c * BC, BC)]], row768_v, bsem
            ).wait()
            pltpu.sync_copy(row768_v, out_bert.at[pl.ds(base + c * BC, BC)])

    return k(users, items, ent0, user_table, news_table, entity_table,
             bert_table)


BLK = 512


def _tc_body(u_ref, n_ref, e_ref, bt_ref, cat_ref, sub_ref,
             catt_ref, subt_ref, wb_ref, bb_ref, wc_ref, bc_ref, o_ref):
    f32 = jnp.float32
    bert = jax.nn.sigmoid(
        jnp.dot(bt_ref[...], wb_ref[...], preferred_element_type=f32)
        + bb_ref[...])
    ids = lax.broadcasted_iota(jnp.int32, (BLK, NCAT), 1)
    cat_oh = (cat_ref[...][:, None] == ids).astype(f32)
    sub_oh = (sub_ref[...][:, None] == ids).astype(f32)
    cat50 = jnp.dot(cat_oh, catt_ref[...], preferred_element_type=f32)
    sub50 = jnp.dot(sub_oh, subt_ref[...], preferred_element_type=f32)
    wc = wc_ref[...]
    z = (jnp.dot(n_ref[...], wc[0:EMB], preferred_element_type=f32)
         + jnp.dot(bert, wc[EMB:2 * EMB], preferred_element_type=f32)
         + jnp.dot(cat50, wc[2 * EMB:3 * EMB], preferred_element_type=f32)
         + jnp.dot(sub50, wc[3 * EMB:4 * EMB], preferred_element_type=f32)
         + jnp.dot(e_ref[...], wc[4 * EMB:5 * EMB], preferred_element_type=f32)
         + bc_ref[...])
    nc = jax.nn.sigmoid(z)
    o_ref[...] = jax.nn.sigmoid(jnp.sum(u_ref[...] * nc, axis=1))


def _tc_compute(user50, news50, ent50, bert768, categories, subcategories,
                cat_table, subcat_table, W_bert, b_bert, W_content, b_content):
    grid = B // BLK
    row_spec = pl.BlockSpec((BLK, EMB), lambda i: (i, 0))
    bert_spec = pl.BlockSpec((BLK, BERT_DIM), lambda i: (i, 0))
    id_spec = pl.BlockSpec((BLK,), lambda i: (i,))
    full = lambda shape: pl.BlockSpec(shape, lambda i: (0,) * len(shape))
    return pl.pallas_call(
        _tc_body,
        grid=(grid,),
        in_specs=[row_spec, row_spec, row_spec, bert_spec, id_spec, id_spec,
                  full((NCAT, EMB)), full((NCAT, EMB)),
                  full((BERT_DIM, EMB)), full((EMB,)),
                  full((5 * EMB, EMB)), full((EMB,))],
        out_specs=pl.BlockSpec((BLK,), lambda i: (i,)),
        out_shape=jax.ShapeDtypeStruct((B,), jnp.float32),
    )(user50, news50, ent50, bert768, categories, subcategories,
      cat_table, subcat_table, W_bert, b_bert, W_content, b_content)


def kernel(users, items, categories, subcategories, entities,
           user_table, news_table, cat_table, subcat_table, entity_table,
           bert_table, W_bert, b_bert, W_content, b_content):
    ent0 = entities[:, 0]
    user50, news50, ent50, bert768 = _sc_gather(
        users, items, ent0, user_table, news_table, entity_table, bert_table)
    return _tc_compute(user50, news50, ent50, bert768,
                       categories, subcategories, cat_table, subcat_table,
                       W_bert, b_bert, W_content, b_content)


# speedup vs baseline: 1.0439x; 1.0439x over previous
import functools

import jax
import jax.numpy as jnp
from jax import lax
from jax.experimental import pallas as pl
from jax.experimental.pallas import tpu as pltpu
from jax.experimental.pallas import tpu_sc as plsc

B = 16384
EMB = 50
BERT_DIM = 768
NCAT = 1000
NW = 32
BPW = B // NW
BC = 64
NBC = BPW // BC


def _sc_gather(users, items, ent0, user_table, news_table, entity_table,
               bert_table):
    mesh = plsc.VectorSubcoreMesh(core_axis_name="c", subcore_axis_name="s")
    out_type = (
        jax.ShapeDtypeStruct((B, EMB), jnp.float32),
        jax.ShapeDtypeStruct((B, EMB), jnp.float32),
        jax.ShapeDtypeStruct((B, EMB), jnp.float32),
        jax.ShapeDtypeStruct((B, BERT_DIM), jnp.float32),
    )

    @functools.partial(
        pl.kernel, mesh=mesh, out_type=out_type,
        scratch_types=[
            pltpu.VMEM((BPW, EMB), jnp.float32),
            pltpu.VMEM((BPW,), jnp.int32),
            pltpu.VMEM((BC, BERT_DIM), jnp.float32),
            pltpu.SemaphoreType.DMA,
            pltpu.SemaphoreType.DMA,
        ],
        compiler_params=pltpu.CompilerParams(needs_layout_passes=False),
    )
    def k(users_h, items_h, ent_h, user_t, news_t, ent_t, bert_t,
          out_user, out_news, out_ent, out_bert,
          rows_v, idx_v, row768_v, sem, bsem):
        wid = lax.axis_index("s") * 2 + lax.axis_index("c")
        base = wid * BPW
        for idx_h, tab, out in ((users_h, user_t, out_user),
                                (items_h, news_t, out_news),
                                (ent_h, ent_t, out_ent)):
            pltpu.sync_copy(idx_h.at[pl.ds(base, BPW)], idx_v)

            def fire(g, _):
                v = idx_v[pl.ds(g * 16, 16)]
                for lane in range(16):
                    s = lax.squeeze(lax.slice(v, (lane,), (lane + 1,)), (0,))
                    pltpu.async_copy(tab.at[pl.ds(s, 1)],
                                     rows_v.at[pl.ds(g * 16 + lane, 1)], sem)
                return 0

            lax.fori_loop(0, BPW // 16, fire, 0)

            def drain(r, _):
                pltpu.make_async_copy(tab.at[pl.ds(0, 1)],
                                      rows_v.at[pl.ds(0, 1)], sem).wait()
                return 0

            lax.fori_loop(0, BPW, drain, 0)
            pltpu.sync_copy(rows_v, out.at[pl.ds(base, BPW)])
        pltpu.sync_copy(items_h.at[pl.ds(base, BPW)], idx_v)
        for c in range(NBC):
            pltpu.async_copy(
                bert_t.at[idx_v.at[pl.ds(c * BC, BC)]], row768_v, bsem
            ).wait()
            pltpu.sync_copy(row768_v, out_bert.at[pl.ds(base + c * BC, BC)])

    return k(users, items, ent0, user_table, news_table, entity_table,
             bert_table)


BLK = 512


def _tc_body(u_ref, n_ref, e_ref, bt_ref, cat_ref, sub_ref,
             catt_ref, subt_ref, wb_ref, bb_ref, wc_ref, bc_ref, o_ref):
    f32 = jnp.float32
    bert = jax.nn.sigmoid(
        jnp.dot(bt_ref[...], wb_ref[...], preferred_element_type=f32)
        + bb_ref[...])
    ids = lax.broadcasted_iota(jnp.int32, (BLK, NCAT), 1)
    cat_oh = (cat_ref[...][:, None] == ids).astype(f32)
    sub_oh = (sub_ref[...][:, None] == ids).astype(f32)
    cat50 = jnp.dot(cat_oh, catt_ref[...], preferred_element_type=f32)
    sub50 = jnp.dot(sub_oh, subt_ref[...], preferred_element_type=f32)
    wc = wc_ref[...]
    z = (jnp.dot(n_ref[...], wc[0:EMB], preferred_element_type=f32)
         + jnp.dot(bert, wc[EMB:2 * EMB], preferred_element_type=f32)
         + jnp.dot(cat50, wc[2 * EMB:3 * EMB], preferred_element_type=f32)
         + jnp.dot(sub50, wc[3 * EMB:4 * EMB], preferred_element_type=f32)
         + jnp.dot(e_ref[...], wc[4 * EMB:5 * EMB], preferred_element_type=f32)
         + bc_ref[...])
    nc = jax.nn.sigmoid(z)
    o_ref[...] = jax.nn.sigmoid(jnp.sum(u_ref[...] * nc, axis=1))


def _tc_compute(user50, news50, ent50, bert768, categories, subcategories,
                cat_table, subcat_table, W_bert, b_bert, W_content, b_content):
    grid = B // BLK
    row_spec = pl.BlockSpec((BLK, EMB), lambda i: (i, 0))
    bert_spec = pl.BlockSpec((BLK, BERT_DIM), lambda i: (i, 0))
    id_spec = pl.BlockSpec((BLK,), lambda i: (i,))
    full = lambda shape: pl.BlockSpec(shape, lambda i: (0,) * len(shape))
    return pl.pallas_call(
        _tc_body,
        grid=(grid,),
        in_specs=[row_spec, row_spec, row_spec, bert_spec, id_spec, id_spec,
                  full((NCAT, EMB)), full((NCAT, EMB)),
                  full((BERT_DIM, EMB)), full((EMB,)),
                  full((5 * EMB, EMB)), full((EMB,))],
        out_specs=pl.BlockSpec((BLK,), lambda i: (i,)),
        out_shape=jax.ShapeDtypeStruct((B,), jnp.float32),
    )(user50, news50, ent50, bert768, categories, subcategories,
      cat_table, subcat_table, W_bert, b_bert, W_content, b_content)


def kernel(users, items, categories, subcategories, entities,
           user_table, news_table, cat_table, subcat_table, entity_table,
           bert_table, W_bert, b_bert, W_content, b_content):
    ent0 = entities[:, 0]
    user50, news50, ent50, bert768 = _sc_gather(
        users, items, ent0, user_table, news_table, entity_table, bert_table)
    return _tc_compute(user50, news50, ent50, bert768,
                       categories, subcategories, cat_table, subcat_table,
                       W_bert, b_bert, W_content, b_content)
